# Initial kernel scaffold; baseline (speedup 1.0000x reference)
import jax
import jax.numpy as jnp
from jax.experimental import pallas as pl


def kernel(score_map, delta_map, anchors):
    raise NotImplementedError("write your pallas kernel here")


# speedup vs baseline: 2.3406x; 2.3406x over previous
import jax
import jax.numpy as jnp
from jax import lax
from jax.experimental import pallas as pl
from jax.experimental.pallas import tpu as pltpu

_B, _H, _W, _K = 4, 48, 48, 9
_N = _H * _W * _K
_MOS = 100
_IOU_T = 0.9
_SCORE_T = 0.9
_PROP_T = 0.5
_NEG_INF = float("-inf")


def _nms_kernel(score_ref, delta_ref, anchor_ref, out_ref):
    tx = delta_ref[0]
    ty = delta_ref[1]
    tw = delta_ref[2]
    th = delta_ref[3]
    a0 = anchor_ref[0:1, :]
    a1 = anchor_ref[1:2, :]
    a2 = anchor_ref[2:3, :]
    a3 = anchor_ref[3:4, :]
    xa = (a0 + a1) * 0.5
    ya = (a2 + a3) * 0.5
    wa = a1 - a0
    ha = a3 - a2
    x = tx * wa + xa
    y = ty * ha + ya
    w = jnp.exp(tw) * wa
    h = jnp.exp(th) * ha
    o_ymax = jnp.minimum(y + h * 0.5, 1.0)
    o_xmin = jnp.maximum(x - w * 0.5, 0.0)
    o_ymin = jnp.maximum(y - h * 0.5, 0.0)
    o_xmax = jnp.minimum(x + w * 0.5, 1.0)
    ymin = jnp.minimum(o_ymin, o_ymax)
    ymax = jnp.maximum(o_ymin, o_ymax)
    xmin = jnp.minimum(o_xmin, o_xmax)
    xmax = jnp.maximum(o_xmin, o_xmax)
    area = (ymax - ymin) * (xmax - xmin)

    sc0 = jnp.where(score_ref[...] > _PROP_T, score_ref[...], _NEG_INF)
    iota = lax.broadcasted_iota(jnp.int32, (_B, _N), 1)

    def body(i, sc):
        best_val = jnp.max(sc, axis=1, keepdims=True)
        eq = sc == best_val
        bidx = jnp.min(jnp.where(eq, iota, _N), axis=1, keepdims=True)
        onehot = iota == bidx

        def sel(v):
            return jnp.sum(jnp.where(onehot, v, 0.0), axis=1, keepdims=True)

        b_ymin = sel(ymin)
        b_ymax = sel(ymax)
        b_xmin = sel(xmin)
        b_xmax = sel(xmax)
        b_area = (b_ymax - b_ymin) * (b_xmax - b_xmin)

        valid = (best_val > _SCORE_T).astype(jnp.float32)
        row = jnp.concatenate(
            [sel(o_ymax) * valid, sel(o_xmin) * valid,
             sel(o_ymin) * valid, sel(o_xmax) * valid],
            axis=1,
        )
        out_ref[pl.ds(i, 1), :, :] = row.reshape(1, _B, 4)

        iy1 = jnp.maximum(b_ymin, ymin)
        iy2 = jnp.minimum(b_ymax, ymax)
        ix1 = jnp.maximum(b_xmin, xmin)
        ix2 = jnp.minimum(b_xmax, xmax)
        inter = jnp.maximum(iy2 - iy1, 0.0) * jnp.maximum(ix2 - ix1, 0.0)
        iou = inter / (b_area + area - inter + 1e-8)
        sc = jnp.where(iou > _IOU_T, _NEG_INF, sc)
        sc = jnp.where(onehot, _NEG_INF, sc)
        return sc

    lax.fori_loop(0, _MOS, body, sc0)


def kernel(score_map, delta_map, anchors):
    scores = score_map.reshape(_B, _N)
    deltas = delta_map.reshape(_B, _N, 4).transpose(2, 0, 1)
    anc = anchors.reshape(_N, 4).T
    out = pl.pallas_call(
        _nms_kernel,
        out_shape=jax.ShapeDtypeStruct((_MOS, _B, 4), jnp.float32),
    )(scores, deltas, anc)
    return out.transpose(1, 0, 2)


# speedup vs baseline: 4.3987x; 1.8793x over previous
import jax
import jax.numpy as jnp
from jax import lax
from jax.experimental import pallas as pl
from jax.experimental.pallas import tpu as pltpu

_B, _H, _W, _K = 4, 48, 48, 9
_N = _H * _W * _K
_S = 8
_C = _N // _S
_MOS = 100
_IOU_T = 0.9
_SCORE_T = 0.9
_PROP_T = 0.5
_NEG_INF = float("-inf")


def _nms_kernel(score_ref, delta_ref, anchor_ref, out_ref):
    tx = delta_ref[0]
    ty = delta_ref[1]
    tw = delta_ref[2]
    th = delta_ref[3]
    a0 = anchor_ref[0:1, :, :]
    a1 = anchor_ref[1:2, :, :]
    a2 = anchor_ref[2:3, :, :]
    a3 = anchor_ref[3:4, :, :]
    xa = (a0 + a1) * 0.5
    ya = (a2 + a3) * 0.5
    wa = a1 - a0
    ha = a3 - a2
    x = tx * wa + xa
    y = ty * ha + ya
    w = jnp.exp(tw) * wa
    h = jnp.exp(th) * ha
    o_ymax = jnp.minimum(y + h * 0.5, 1.0)
    o_xmin = jnp.maximum(x - w * 0.5, 0.0)
    o_ymin = jnp.maximum(y - h * 0.5, 0.0)
    o_xmax = jnp.minimum(x + w * 0.5, 1.0)
    ymin = jnp.minimum(o_ymin, o_ymax)
    ymax = jnp.maximum(o_ymin, o_ymax)
    xmin = jnp.minimum(o_xmin, o_xmax)
    xmax = jnp.maximum(o_xmin, o_xmax)
    area = (ymax - ymin) * (xmax - xmin)

    sc0 = jnp.where(score_ref[...] > _PROP_T, score_ref[...], _NEG_INF)
    iota = (lax.broadcasted_iota(jnp.int32, (_B, _S, _C), 1) * _C
            + lax.broadcasted_iota(jnp.int32, (_B, _S, _C), 2))

    def body(i, sc):
        best_val = jnp.max(sc, axis=(1, 2), keepdims=True)
        eq = sc == best_val
        bidx = jnp.min(jnp.where(eq, iota, _N), axis=(1, 2), keepdims=True)
        onehot = iota == bidx

        def sel(v):
            return jnp.sum(jnp.where(onehot, v, 0.0), axis=(1, 2), keepdims=True)

        b_oymax = sel(o_ymax)
        b_oxmin = sel(o_xmin)
        b_oymin = sel(o_ymin)
        b_oxmax = sel(o_xmax)
        b_ymin = jnp.minimum(b_oymin, b_oymax)
        b_ymax = jnp.maximum(b_oymin, b_oymax)
        b_xmin = jnp.minimum(b_oxmin, b_oxmax)
        b_xmax = jnp.maximum(b_oxmin, b_oxmax)
        b_area = (b_ymax - b_ymin) * (b_xmax - b_xmin)

        valid = (best_val > _SCORE_T).astype(jnp.float32)
        row = jnp.concatenate(
            [b_oymax * valid, b_oxmin * valid, b_oymin * valid, b_oxmax * valid],
            axis=2,
        )
        out_ref[pl.ds(i, 1), :, :] = row.reshape(1, _B, 4)

        iy1 = jnp.maximum(b_ymin, ymin)
        iy2 = jnp.minimum(b_ymax, ymax)
        ix1 = jnp.maximum(b_xmin, xmin)
        ix2 = jnp.minimum(b_xmax, xmax)
        inter = jnp.maximum(iy2 - iy1, 0.0) * jnp.maximum(ix2 - ix1, 0.0)
        iou = inter / (b_area + area - inter + 1e-8)
        sc = jnp.where(jnp.logical_or(iou > _IOU_T, onehot), _NEG_INF, sc)
        return sc

    lax.fori_loop(0, _MOS, body, sc0)


def kernel(score_map, delta_map, anchors):
    scores = score_map.reshape(_B, _S, _C)
    deltas = delta_map.reshape(_B, _N, 4).transpose(2, 0, 1).reshape(4, _B, _S, _C)
    anc = anchors.reshape(_N, 4).T.reshape(4, _S, _C)
    out = pl.pallas_call(
        _nms_kernel,
        out_shape=jax.ShapeDtypeStruct((_MOS, _B, 4), jnp.float32),
    )(scores, deltas, anc)
    return out.transpose(1, 0, 2)
